# Initial kernel scaffold; baseline (speedup 1.0000x reference)
import functools

import jax
import jax.numpy as jnp
from jax import lax
from jax.experimental import pallas as pl
from jax.experimental.pallas import tpu as pltpu
from jax.experimental.pallas import tpu_sc as plsc

N = 10000
D = 128
C = 40
K = 10
ALPHA = 0.1

NP = 10240
CP = 48
DUMMY = 10000

NTILES = 32
CHUNK = 128
CHUNKS_PER_TILE = 79
EP = NTILES * CHUNKS_PER_TILE * CHUNK

ROWS_PER_TILE = NP // 16


def _sc_body(g_hbm, row_hbm, col_hbm, zero_hbm, out_hbm,
             row_v, col_v, rows_v, s_sh, sem):
    c = lax.axis_index("c")
    s = lax.axis_index("s")
    t = c * 16 + s
    r0 = s * ROWS_PER_TILE

    pltpu.sync_copy(zero_hbm.at[pl.ds(r0, ROWS_PER_TILE)],
                    s_sh.at[pl.ds(r0, ROWS_PER_TILE)])
    plsc.subcore_barrier()

    def step(j, carry):
        pltpu.sync_copy(row_hbm.at[t, j], row_v)
        pltpu.sync_copy(col_hbm.at[t, j], col_v)
        pltpu.async_copy(g_hbm.at[row_v], rows_v, sem).wait()
        pltpu.sync_copy(rows_v, s_sh.at[col_v], add=True)
        return carry

    lax.fori_loop(0, CHUNKS_PER_TILE, step, 0)

    plsc.subcore_barrier()
    pltpu.sync_copy(s_sh.at[pl.ds(r0, ROWS_PER_TILE)],
                    out_hbm.at[c, pl.ds(r0, ROWS_PER_TILE)])


_sc_propagate = functools.partial(
    pl.kernel,
    out_type=jax.ShapeDtypeStruct((2, NP, CP), jnp.float32),
    mesh=plsc.VectorSubcoreMesh(core_axis_name="c", subcore_axis_name="s"),
    scratch_types=[
        pltpu.VMEM((CHUNK,), jnp.int32),
        pltpu.VMEM((CHUNK,), jnp.int32),
        pltpu.VMEM((CHUNK, CP), jnp.float32),
        pltpu.VMEM_SHARED((NP, CP), jnp.float32),
        pltpu.SemaphoreType.DMA,
    ],
)(_sc_body)


_BLK = 128
_GRID = NP // _BLK

def _row_spec():
    return pl.BlockSpec((_BLK, CP), lambda i: (i, 0))


def _lin_body(x_ref, w_ref, b_ref, o_ref):
    acc = jnp.dot(x_ref[...], w_ref[...], preferred_element_type=jnp.float32)
    o_ref[...] = jnp.maximum(acc + b_ref[...], 0.0)


def _tc_linear(x_pad, w_pad, b_pad):
    return pl.pallas_call(
        _lin_body,
        grid=(_GRID,),
        in_specs=[
            pl.BlockSpec((_BLK, D), lambda i: (i, 0)),
            pl.BlockSpec((D, CP), lambda i: (0, 0)),
            pl.BlockSpec((1, CP), lambda i: (0, 0)),
        ],
        out_specs=_row_spec(),
        out_shape=jax.ShapeDtypeStruct((NP, CP), jnp.float32),
    )(x_pad, w_pad, b_pad)


def _init_body(s0_ref, s1_ref, h0_ref, g_ref, c0_ref, c1_ref, dinv_ref):
    deg = s0_ref[...] + s1_ref[...] + 1.0
    dinv = lax.rsqrt(deg)
    g = dinv * h0_ref[...]
    g_ref[...] = g
    c0_ref[...] = ALPHA * g
    c1_ref[...] = (1.0 - ALPHA) * dinv * dinv
    dinv_ref[...] = dinv


def _tc_init(s_deg, h0):
    return pl.pallas_call(
        _init_body,
        grid=(_GRID,),
        in_specs=[_row_spec(), _row_spec(), _row_spec()],
        out_specs=[_row_spec(), _row_spec(), _row_spec(), _row_spec()],
        out_shape=[jax.ShapeDtypeStruct((NP, CP), jnp.float32)] * 4,
    )(s_deg[0], s_deg[1], h0)


def _combine_body(s0_ref, s1_ref, g_ref, c0_ref, c1_ref, o_ref):
    o_ref[...] = (c1_ref[...] * (s0_ref[...] + s1_ref[...] + g_ref[...])
                  + c0_ref[...])


def _tc_combine(sp, g, c0, c1):
    return pl.pallas_call(
        _combine_body,
        grid=(_GRID,),
        in_specs=[_row_spec()] * 5,
        out_specs=_row_spec(),
        out_shape=jax.ShapeDtypeStruct((NP, CP), jnp.float32),
    )(sp[0], sp[1], g, c0, c1)


def _final_body(s0_ref, s1_ref, g_ref, dinv_ref, h0_ref, o_ref):
    h = ((1.0 - ALPHA) * dinv_ref[...]
         * (s0_ref[...] + s1_ref[...] + g_ref[...])
         + ALPHA * h0_ref[...])
    col = lax.broadcasted_iota(jnp.int32, (_BLK, CP), 1)
    hm = jnp.where(col < C, h, -1e30)
    m = jnp.max(hm, axis=1, keepdims=True)
    e = jnp.exp(hm - m)
    lse = jnp.log(jnp.sum(e, axis=1, keepdims=True)) + m
    o_ref[...] = hm - lse


def _tc_final(sp, g, dinv, h0):
    return pl.pallas_call(
        _final_body,
        grid=(_GRID,),
        in_specs=[_row_spec()] * 5,
        out_specs=_row_spec(),
        out_shape=jax.ShapeDtypeStruct((NP, CP), jnp.float32),
    )(sp[0], sp[1], g, dinv, h0)


def kernel(x, edge_index, W, b):
    x_pad = jnp.pad(x, ((0, NP - N), (0, 0)))
    w_pad = jnp.pad(W, ((0, 0), (0, CP - C)))
    b_pad = jnp.pad(b, (0, CP - C)).reshape(1, CP)

    row = jnp.pad(edge_index[0], (0, EP - edge_index.shape[1]),
                  constant_values=DUMMY).reshape(NTILES, CHUNKS_PER_TILE, CHUNK)
    col = jnp.pad(edge_index[1], (0, EP - edge_index.shape[1]),
                  constant_values=DUMMY).reshape(NTILES, CHUNKS_PER_TILE, CHUNK)

    zero_nc = jnp.zeros((NP, CP), jnp.float32)
    ones_nc = jnp.ones((NP, CP), jnp.float32)

    h0 = _tc_linear(x_pad, w_pad, b_pad)
    s_deg = _sc_propagate(ones_nc, row, col, zero_nc)
    g, c0, c1, dinv = _tc_init(s_deg, h0)

    for _ in range(K - 1):
        sp = _sc_propagate(g, row, col, zero_nc)
        g = _tc_combine(sp, g, c0, c1)
    sp = _sc_propagate(g, row, col, zero_nc)

    z = _tc_final(sp, g, dinv, h0)
    return z[:N, :C]


# speedup vs baseline: 8.1579x; 8.1579x over previous
import functools

import jax
import jax.numpy as jnp
from jax import lax
from jax.experimental import pallas as pl
from jax.experimental.pallas import tpu as pltpu
from jax.experimental.pallas import tpu_sc as plsc

N = 10000
D = 128
C = 40
K = 10
ALPHA = 0.1

NP = 10240
CP = 48
DUMMY = 10000

NTILES = 32
CHUNK = 128
CHUNKS_PER_TILE = 79
EP = NTILES * CHUNKS_PER_TILE * CHUNK

ROWS_PER_TILE = NP // 16


def _sc_body(g_hbm, row_hbm, col_hbm, zero_hbm, out_hbm,
             row_v, col_v, rows_v, s_sh, sem):
    c = lax.axis_index("c")
    s = lax.axis_index("s")
    t = c * 16 + s
    r0 = s * ROWS_PER_TILE

    pltpu.sync_copy(zero_hbm.at[pl.ds(r0, ROWS_PER_TILE)],
                    s_sh.at[pl.ds(r0, ROWS_PER_TILE)])
    plsc.subcore_barrier()

    def step(j, carry):
        pltpu.sync_copy(row_hbm.at[t, j], row_v)
        pltpu.sync_copy(col_hbm.at[t, j], col_v)
        pltpu.async_copy(g_hbm.at[row_v], rows_v, sem).wait()
        pltpu.sync_copy(rows_v, s_sh.at[col_v], add=True)
        return carry

    lax.fori_loop(0, CHUNKS_PER_TILE, step, 0)

    plsc.subcore_barrier()
    pltpu.sync_copy(s_sh.at[pl.ds(r0, ROWS_PER_TILE)],
                    out_hbm.at[c, pl.ds(r0, ROWS_PER_TILE)])


_sc_propagate = functools.partial(
    pl.kernel,
    out_type=jax.ShapeDtypeStruct((2, NP, CP), jnp.float32),
    mesh=plsc.VectorSubcoreMesh(core_axis_name="c", subcore_axis_name="s"),
    scratch_types=[
        pltpu.VMEM((CHUNK,), jnp.int32),
        pltpu.VMEM((CHUNK,), jnp.int32),
        pltpu.VMEM((CHUNK, CP), jnp.float32),
        pltpu.VMEM_SHARED((NP, CP), jnp.float32),
        pltpu.SemaphoreType.DMA,
    ],
    compiler_params=pltpu.CompilerParams(use_tc_tiling_on_sc=False),
)(_sc_body)


_BLK = 128
_GRID = NP // _BLK

def _row_spec():
    return pl.BlockSpec((_BLK, CP), lambda i: (i, 0))


def _lin_body(x_ref, w_ref, b_ref, o_ref):
    acc = jnp.dot(x_ref[...], w_ref[...], preferred_element_type=jnp.float32)
    o_ref[...] = jnp.maximum(acc + b_ref[...], 0.0)


def _tc_linear(x_pad, w_pad, b_pad):
    return pl.pallas_call(
        _lin_body,
        grid=(_GRID,),
        in_specs=[
            pl.BlockSpec((_BLK, D), lambda i: (i, 0)),
            pl.BlockSpec((D, CP), lambda i: (0, 0)),
            pl.BlockSpec((1, CP), lambda i: (0, 0)),
        ],
        out_specs=_row_spec(),
        out_shape=jax.ShapeDtypeStruct((NP, CP), jnp.float32),
    )(x_pad, w_pad, b_pad)


def _init_body(s0_ref, s1_ref, h0_ref, g_ref, c0_ref, c1_ref, dinv_ref):
    deg = s0_ref[...] + s1_ref[...] + 1.0
    dinv = lax.rsqrt(deg)
    g = dinv * h0_ref[...]
    g_ref[...] = g
    c0_ref[...] = ALPHA * g
    c1_ref[...] = (1.0 - ALPHA) * dinv * dinv
    dinv_ref[...] = dinv


def _tc_init(s_deg, h0):
    return pl.pallas_call(
        _init_body,
        grid=(_GRID,),
        in_specs=[_row_spec(), _row_spec(), _row_spec()],
        out_specs=[_row_spec(), _row_spec(), _row_spec(), _row_spec()],
        out_shape=[jax.ShapeDtypeStruct((NP, CP), jnp.float32)] * 4,
    )(s_deg[0], s_deg[1], h0)


def _combine_body(s0_ref, s1_ref, g_ref, c0_ref, c1_ref, o_ref):
    o_ref[...] = (c1_ref[...] * (s0_ref[...] + s1_ref[...] + g_ref[...])
                  + c0_ref[...])


def _tc_combine(sp, g, c0, c1):
    return pl.pallas_call(
        _combine_body,
        grid=(_GRID,),
        in_specs=[_row_spec()] * 5,
        out_specs=_row_spec(),
        out_shape=jax.ShapeDtypeStruct((NP, CP), jnp.float32),
    )(sp[0], sp[1], g, c0, c1)


def _final_body(s0_ref, s1_ref, g_ref, dinv_ref, h0_ref, o_ref):
    h = ((1.0 - ALPHA) * dinv_ref[...]
         * (s0_ref[...] + s1_ref[...] + g_ref[...])
         + ALPHA * h0_ref[...])
    col = lax.broadcasted_iota(jnp.int32, (_BLK, CP), 1)
    hm = jnp.where(col < C, h, -1e30)
    m = jnp.max(hm, axis=1, keepdims=True)
    e = jnp.exp(hm - m)
    lse = jnp.log(jnp.sum(e, axis=1, keepdims=True)) + m
    o_ref[...] = hm - lse


def _tc_final(sp, g, dinv, h0):
    return pl.pallas_call(
        _final_body,
        grid=(_GRID,),
        in_specs=[_row_spec()] * 5,
        out_specs=_row_spec(),
        out_shape=jax.ShapeDtypeStruct((NP, CP), jnp.float32),
    )(sp[0], sp[1], g, dinv, h0)


def kernel(x, edge_index, W, b):
    x_pad = jnp.pad(x, ((0, NP - N), (0, 0)))
    w_pad = jnp.pad(W, ((0, 0), (0, CP - C)))
    b_pad = jnp.pad(b, (0, CP - C)).reshape(1, CP)

    row = jnp.pad(edge_index[0], (0, EP - edge_index.shape[1]),
                  constant_values=DUMMY).reshape(NTILES, CHUNKS_PER_TILE, CHUNK)
    col = jnp.pad(edge_index[1], (0, EP - edge_index.shape[1]),
                  constant_values=DUMMY).reshape(NTILES, CHUNKS_PER_TILE, CHUNK)

    zero_nc = jnp.zeros((NP, CP), jnp.float32)
    ones_nc = jnp.ones((NP, CP), jnp.float32)

    h0 = _tc_linear(x_pad, w_pad, b_pad)
    s_deg = _sc_propagate(ones_nc, row, col, zero_nc)
    g, c0, c1, dinv = _tc_init(s_deg, h0)

    for _ in range(K - 1):
        sp = _sc_propagate(g, row, col, zero_nc)
        g = _tc_combine(sp, g, c0, c1)
    sp = _sc_propagate(g, row, col, zero_nc)

    z = _tc_final(sp, g, dinv, h0)
    return z[:N, :C]


# speedup vs baseline: 8.8131x; 1.0803x over previous
import functools

import jax
import jax.numpy as jnp
from jax import lax
from jax.experimental import pallas as pl
from jax.experimental.pallas import tpu as pltpu
from jax.experimental.pallas import tpu_sc as plsc

N = 10000
D = 128
C = 40
K = 10
ALPHA = 0.1

NP = 10240
CP = 48
DUMMY = 10000

NTILES = 32
CHUNK = 128
CHUNKS_PER_TILE = 80
EP = NTILES * CHUNKS_PER_TILE * CHUNK

NBUF = 10
LOOKAHEAD = 5

ROWS_PER_TILE = NP // 16


def _sc_body(g_hbm, row_hbm, col_hbm, zero_hbm, out_hbm,
             row_all, col_all, rows, s_sh, gsem, ssem):
    c = lax.axis_index("c")
    s = lax.axis_index("s")
    t = c * 16 + s
    r0 = s * ROWS_PER_TILE

    pltpu.sync_copy(row_hbm.at[t], row_all)
    pltpu.sync_copy(col_hbm.at[t], col_all)
    pltpu.sync_copy(zero_hbm.at[pl.ds(r0, ROWS_PER_TILE)],
                    s_sh.at[pl.ds(r0, ROWS_PER_TILE)])
    plsc.subcore_barrier()

    def gather_start(j, b):
        pltpu.async_copy(g_hbm.at[row_all.at[j]], rows.at[b], gsem.at[b])

    def gather_wait(b):
        pltpu.make_async_copy(g_hbm.at[row_all.at[b]], rows.at[b],
                              gsem.at[b]).wait()

    def scatter_start(j, b):
        pltpu.async_copy(rows.at[b], s_sh.at[col_all.at[j]], ssem.at[b],
                         add=True)

    def scatter_wait(b):
        pltpu.make_async_copy(rows.at[b], s_sh.at[col_all.at[b]],
                              ssem.at[b]).wait()

    for b in range(LOOKAHEAD):
        gather_start(b, b)

    def outer(i, carry):
        jo = i * NBUF
        for b in range(NBUF):
            j = jo + b
            bg = (b + LOOKAHEAD) % NBUF
            gather_wait(b)
            scatter_start(j, b)

            @pl.when(j >= LOOKAHEAD)
            def _():
                scatter_wait(bg)

            @pl.when(j + LOOKAHEAD < CHUNKS_PER_TILE)
            def _():
                gather_start(j + LOOKAHEAD, bg)
        return carry

    lax.fori_loop(0, CHUNKS_PER_TILE // NBUF, outer, 0)
    for b in range(NBUF - LOOKAHEAD, NBUF):
        scatter_wait(b)

    plsc.subcore_barrier()
    pltpu.sync_copy(s_sh.at[pl.ds(r0, ROWS_PER_TILE)],
                    out_hbm.at[c, pl.ds(r0, ROWS_PER_TILE)])


_sc_propagate = functools.partial(
    pl.kernel,
    out_type=jax.ShapeDtypeStruct((2, NP, CP), jnp.float32),
    mesh=plsc.VectorSubcoreMesh(core_axis_name="c", subcore_axis_name="s"),
    scratch_types=[
        pltpu.VMEM((CHUNKS_PER_TILE, CHUNK), jnp.int32),
        pltpu.VMEM((CHUNKS_PER_TILE, CHUNK), jnp.int32),
        pltpu.VMEM((NBUF, CHUNK, CP), jnp.float32),
        pltpu.VMEM_SHARED((NP, CP), jnp.float32),
        pltpu.SemaphoreType.DMA((NBUF,)),
        pltpu.SemaphoreType.DMA((NBUF,)),
    ],
    compiler_params=pltpu.CompilerParams(use_tc_tiling_on_sc=False),
)(_sc_body)


_BLK = 128
_GRID = NP // _BLK

def _row_spec():
    return pl.BlockSpec((_BLK, CP), lambda i: (i, 0))


def _lin_body(x_ref, w_ref, b_ref, o_ref):
    acc = jnp.dot(x_ref[...], w_ref[...], preferred_element_type=jnp.float32)
    o_ref[...] = jnp.maximum(acc + b_ref[...], 0.0)


def _tc_linear(x_pad, w_pad, b_pad):
    return pl.pallas_call(
        _lin_body,
        grid=(_GRID,),
        in_specs=[
            pl.BlockSpec((_BLK, D), lambda i: (i, 0)),
            pl.BlockSpec((D, CP), lambda i: (0, 0)),
            pl.BlockSpec((1, CP), lambda i: (0, 0)),
        ],
        out_specs=_row_spec(),
        out_shape=jax.ShapeDtypeStruct((NP, CP), jnp.float32),
    )(x_pad, w_pad, b_pad)


def _init_body(s0_ref, s1_ref, h0_ref, g_ref, c0_ref, c1_ref, dinv_ref):
    deg = s0_ref[...] + s1_ref[...] + 1.0
    dinv = lax.rsqrt(deg)
    g = dinv * h0_ref[...]
    g_ref[...] = g
    c0_ref[...] = ALPHA * g
    c1_ref[...] = (1.0 - ALPHA) * dinv * dinv
    dinv_ref[...] = dinv


def _tc_init(s_deg, h0):
    return pl.pallas_call(
        _init_body,
        grid=(_GRID,),
        in_specs=[_row_spec(), _row_spec(), _row_spec()],
        out_specs=[_row_spec(), _row_spec(), _row_spec(), _row_spec()],
        out_shape=[jax.ShapeDtypeStruct((NP, CP), jnp.float32)] * 4,
    )(s_deg[0], s_deg[1], h0)


def _combine_body(s0_ref, s1_ref, g_ref, c0_ref, c1_ref, o_ref):
    o_ref[...] = (c1_ref[...] * (s0_ref[...] + s1_ref[...] + g_ref[...])
                  + c0_ref[...])


def _tc_combine(sp, g, c0, c1):
    return pl.pallas_call(
        _combine_body,
        grid=(_GRID,),
        in_specs=[_row_spec()] * 5,
        out_specs=_row_spec(),
        out_shape=jax.ShapeDtypeStruct((NP, CP), jnp.float32),
    )(sp[0], sp[1], g, c0, c1)


def _final_body(s0_ref, s1_ref, g_ref, dinv_ref, h0_ref, o_ref):
    h = ((1.0 - ALPHA) * dinv_ref[...]
         * (s0_ref[...] + s1_ref[...] + g_ref[...])
         + ALPHA * h0_ref[...])
    col = lax.broadcasted_iota(jnp.int32, (_BLK, CP), 1)
    hm = jnp.where(col < C, h, -1e30)
    m = jnp.max(hm, axis=1, keepdims=True)
    e = jnp.exp(hm - m)
    lse = jnp.log(jnp.sum(e, axis=1, keepdims=True)) + m
    o_ref[...] = hm - lse


def _tc_final(sp, g, dinv, h0):
    return pl.pallas_call(
        _final_body,
        grid=(_GRID,),
        in_specs=[_row_spec()] * 5,
        out_specs=_row_spec(),
        out_shape=jax.ShapeDtypeStruct((NP, CP), jnp.float32),
    )(sp[0], sp[1], g, dinv, h0)


def kernel(x, edge_index, W, b):
    x_pad = jnp.pad(x, ((0, NP - N), (0, 0)))
    w_pad = jnp.pad(W, ((0, 0), (0, CP - C)))
    b_pad = jnp.pad(b, (0, CP - C)).reshape(1, CP)

    row = jnp.pad(edge_index[0], (0, EP - edge_index.shape[1]),
                  constant_values=DUMMY).reshape(NTILES, CHUNKS_PER_TILE, CHUNK)
    col = jnp.pad(edge_index[1], (0, EP - edge_index.shape[1]),
                  constant_values=DUMMY).reshape(NTILES, CHUNKS_PER_TILE, CHUNK)

    zero_nc = jnp.zeros((NP, CP), jnp.float32)
    ones_nc = jnp.ones((NP, CP), jnp.float32)

    h0 = _tc_linear(x_pad, w_pad, b_pad)
    s_deg = _sc_propagate(ones_nc, row, col, zero_nc)
    g, c0, c1, dinv = _tc_init(s_deg, h0)

    for _ in range(K - 1):
        sp = _sc_propagate(g, row, col, zero_nc)
        g = _tc_combine(sp, g, c0, c1)
    sp = _sc_propagate(g, row, col, zero_nc)

    z = _tc_final(sp, g, dinv, h0)
    return z[:N, :C]


# speedup vs baseline: 19.4072x; 2.2021x over previous
import functools

import jax
import jax.numpy as jnp
from jax import lax
from jax.experimental import pallas as pl
from jax.experimental.pallas import tpu as pltpu
from jax.experimental.pallas import tpu_sc as plsc

N = 10000
D = 128
C = 40
K = 10
ALPHA = 0.1

NP = 10112
CP = 48
DUMMY = 10000

NTILES = 32
CHUNK = 128
CHUNKS_PER_TILE = 80
EP = NTILES * CHUNKS_PER_TILE * CHUNK

NBUF = 8
LOOKAHEAD = 4

ROWS_PER_TILE = NP // 16


def _sc_body(g_hbm, row_hbm, col_hbm, zero_hbm, out_hbm,
             row_all, col_all, rows, s_sh, g_sh, gsem, ssem):
    c = lax.axis_index("c")
    s = lax.axis_index("s")
    t = c * 16 + s
    r0 = s * ROWS_PER_TILE

    pltpu.sync_copy(row_hbm.at[t], row_all)
    pltpu.sync_copy(col_hbm.at[t], col_all)
    pltpu.sync_copy(g_hbm.at[pl.ds(r0, ROWS_PER_TILE)],
                    g_sh.at[pl.ds(r0, ROWS_PER_TILE)])
    pltpu.sync_copy(zero_hbm.at[pl.ds(r0, ROWS_PER_TILE)],
                    s_sh.at[pl.ds(r0, ROWS_PER_TILE)])
    plsc.subcore_barrier()

    def gather_start(j, b):
        pltpu.async_copy(g_sh.at[row_all.at[j]], rows.at[b], gsem.at[b])

    def gather_wait(b):
        pltpu.make_async_copy(g_sh.at[row_all.at[b]], rows.at[b],
                              gsem.at[b]).wait()

    def scatter_start(j, b):
        pltpu.async_copy(rows.at[b], s_sh.at[col_all.at[j]], ssem.at[b],
                         add=True)

    def scatter_wait(b):
        pltpu.make_async_copy(rows.at[b], s_sh.at[col_all.at[b]],
                              ssem.at[b]).wait()

    for b in range(LOOKAHEAD):
        gather_start(b, b)

    def outer(i, carry):
        jo = i * NBUF
        for b in range(NBUF):
            j = jo + b
            bg = (b + LOOKAHEAD) % NBUF
            gather_wait(b)
            scatter_start(j, b)

            @pl.when(j >= LOOKAHEAD)
            def _():
                scatter_wait(bg)

            @pl.when(j + LOOKAHEAD < CHUNKS_PER_TILE)
            def _():
                gather_start(j + LOOKAHEAD, bg)
        return carry

    lax.fori_loop(0, CHUNKS_PER_TILE // NBUF, outer, 0)
    for b in range(NBUF - LOOKAHEAD, NBUF):
        scatter_wait(b)

    plsc.subcore_barrier()
    pltpu.sync_copy(s_sh.at[pl.ds(r0, ROWS_PER_TILE)],
                    out_hbm.at[c, pl.ds(r0, ROWS_PER_TILE)])


_sc_propagate = functools.partial(
    pl.kernel,
    out_type=jax.ShapeDtypeStruct((2, NP, CP), jnp.float32),
    mesh=plsc.VectorSubcoreMesh(core_axis_name="c", subcore_axis_name="s"),
    scratch_types=[
        pltpu.VMEM((CHUNKS_PER_TILE, CHUNK), jnp.int32),
        pltpu.VMEM((CHUNKS_PER_TILE, CHUNK), jnp.int32),
        pltpu.VMEM((NBUF, CHUNK, CP), jnp.float32),
        pltpu.VMEM_SHARED((NP, CP), jnp.float32),
        pltpu.VMEM_SHARED((NP, CP), jnp.float32),
        pltpu.SemaphoreType.DMA((NBUF,)),
        pltpu.SemaphoreType.DMA((NBUF,)),
    ],
    compiler_params=pltpu.CompilerParams(use_tc_tiling_on_sc=False),
)(_sc_body)


_BLK = 128
_GRID = NP // _BLK

def _row_spec():
    return pl.BlockSpec((_BLK, CP), lambda i: (i, 0))


def _lin_body(x_ref, w_ref, b_ref, o_ref):
    acc = jnp.dot(x_ref[...], w_ref[...], preferred_element_type=jnp.float32)
    o_ref[...] = jnp.maximum(acc + b_ref[...], 0.0)


def _tc_linear(x_pad, w_pad, b_pad):
    return pl.pallas_call(
        _lin_body,
        grid=(_GRID,),
        in_specs=[
            pl.BlockSpec((_BLK, D), lambda i: (i, 0)),
            pl.BlockSpec((D, CP), lambda i: (0, 0)),
            pl.BlockSpec((1, CP), lambda i: (0, 0)),
        ],
        out_specs=_row_spec(),
        out_shape=jax.ShapeDtypeStruct((NP, CP), jnp.float32),
    )(x_pad, w_pad, b_pad)


def _init_body(s0_ref, s1_ref, h0_ref, g_ref, c0_ref, c1_ref, dinv_ref):
    deg = s0_ref[...] + s1_ref[...] + 1.0
    dinv = lax.rsqrt(deg)
    g = dinv * h0_ref[...]
    g_ref[...] = g
    c0_ref[...] = ALPHA * g
    c1_ref[...] = (1.0 - ALPHA) * dinv * dinv
    dinv_ref[...] = dinv


def _tc_init(s_deg, h0):
    return pl.pallas_call(
        _init_body,
        grid=(_GRID,),
        in_specs=[_row_spec(), _row_spec(), _row_spec()],
        out_specs=[_row_spec(), _row_spec(), _row_spec(), _row_spec()],
        out_shape=[jax.ShapeDtypeStruct((NP, CP), jnp.float32)] * 4,
    )(s_deg[0], s_deg[1], h0)


def _combine_body(s0_ref, s1_ref, g_ref, c0_ref, c1_ref, o_ref):
    o_ref[...] = (c1_ref[...] * (s0_ref[...] + s1_ref[...] + g_ref[...])
                  + c0_ref[...])


def _tc_combine(sp, g, c0, c1):
    return pl.pallas_call(
        _combine_body,
        grid=(_GRID,),
        in_specs=[_row_spec()] * 5,
        out_specs=_row_spec(),
        out_shape=jax.ShapeDtypeStruct((NP, CP), jnp.float32),
    )(sp[0], sp[1], g, c0, c1)


def _final_body(s0_ref, s1_ref, g_ref, dinv_ref, h0_ref, o_ref):
    h = ((1.0 - ALPHA) * dinv_ref[...]
         * (s0_ref[...] + s1_ref[...] + g_ref[...])
         + ALPHA * h0_ref[...])
    col = lax.broadcasted_iota(jnp.int32, (_BLK, CP), 1)
    hm = jnp.where(col < C, h, -1e30)
    m = jnp.max(hm, axis=1, keepdims=True)
    e = jnp.exp(hm - m)
    lse = jnp.log(jnp.sum(e, axis=1, keepdims=True)) + m
    o_ref[...] = hm - lse


def _tc_final(sp, g, dinv, h0):
    return pl.pallas_call(
        _final_body,
        grid=(_GRID,),
        in_specs=[_row_spec()] * 5,
        out_specs=_row_spec(),
        out_shape=jax.ShapeDtypeStruct((NP, CP), jnp.float32),
    )(sp[0], sp[1], g, dinv, h0)


def kernel(x, edge_index, W, b):
    x_pad = jnp.pad(x, ((0, NP - N), (0, 0)))
    w_pad = jnp.pad(W, ((0, 0), (0, CP - C)))
    b_pad = jnp.pad(b, (0, CP - C)).reshape(1, CP)

    row = jnp.pad(edge_index[0], (0, EP - edge_index.shape[1]),
                  constant_values=DUMMY).reshape(NTILES, CHUNKS_PER_TILE, CHUNK)
    col = jnp.pad(edge_index[1], (0, EP - edge_index.shape[1]),
                  constant_values=DUMMY).reshape(NTILES, CHUNKS_PER_TILE, CHUNK)

    zero_nc = jnp.zeros((NP, CP), jnp.float32)
    ones_nc = jnp.ones((NP, CP), jnp.float32)

    h0 = _tc_linear(x_pad, w_pad, b_pad)
    s_deg = _sc_propagate(ones_nc, row, col, zero_nc)
    g, c0, c1, dinv = _tc_init(s_deg, h0)

    for _ in range(K - 1):
        sp = _sc_propagate(g, row, col, zero_nc)
        g = _tc_combine(sp, g, c0, c1)
    sp = _sc_propagate(g, row, col, zero_nc)

    z = _tc_final(sp, g, dinv, h0)
    return z[:N, :C]


# speedup vs baseline: 31.8991x; 1.6437x over previous
import functools

import jax
import jax.numpy as jnp
from jax import lax
from jax.experimental import pallas as pl
from jax.experimental.pallas import tpu as pltpu
from jax.experimental.pallas import tpu_sc as plsc

N = 10000
D = 128
C = 40
K = 10
ALPHA = 0.1

NP = 10112
CP = 48
CH = CP // 2
DUMMY = 10000

CHUNK = 128
EP = 327680

DEG_TILES = 32
DEG_CPT = 80
LOOP_CPT = 160

NBUF = 8
LOOKAHEAD = 4

RPT = NP // 16
CMB = 158
NCMB = RPT // CMB


def _sc_deg_body(col_hbm, ones_hbm, zero_hbm, out_hbm,
                 col_all, ones_v, s_sh, ssem):
    c = lax.axis_index("c")
    sid = lax.axis_index("s")
    t = c * 16 + sid
    r0 = sid * RPT

    pltpu.sync_copy(col_hbm.at[t], col_all)
    pltpu.sync_copy(ones_hbm, ones_v)
    pltpu.sync_copy(zero_hbm.at[pl.ds(r0, RPT)], s_sh.at[pl.ds(r0, RPT)])
    plsc.subcore_barrier()

    def fire(j):
        pltpu.async_copy(ones_v, s_sh.at[col_all.at[j]], ssem, add=True)

    def drain_one():
        pltpu.make_async_copy(ones_v, s_sh.at[col_all.at[0]], ssem).wait()

    for b in range(NBUF):
        fire(b)

    def outer(i, carry):
        jo = i * NBUF
        for b in range(NBUF):
            drain_one()

            @pl.when(jo + NBUF + b < DEG_CPT)
            def _():
                fire(jo + NBUF + b)
        return carry

    lax.fori_loop(0, DEG_CPT // NBUF, outer, 0)

    plsc.subcore_barrier()
    pltpu.sync_copy(s_sh.at[pl.ds(r0, RPT)], out_hbm.at[c, pl.ds(r0, RPT)])


_sc_degrees = functools.partial(
    pl.kernel,
    out_type=jax.ShapeDtypeStruct((2, NP, 16), jnp.float32),
    mesh=plsc.VectorSubcoreMesh(core_axis_name="c", subcore_axis_name="s"),
    scratch_types=[
        pltpu.VMEM((DEG_CPT, CHUNK), jnp.int32),
        pltpu.VMEM((CHUNK, 16), jnp.float32),
        pltpu.VMEM_SHARED((NP, 16), jnp.float32),
        pltpu.SemaphoreType.DMA,
    ],
    compiler_params=pltpu.CompilerParams(use_tc_tiling_on_sc=False),
)(_sc_deg_body)


def _sc_loop_body(g0_hbm, c0_hbm, c1_hbm, row_hbm, col_hbm, zero_hbm,
                  g_out,
                  row_all, col_all, rows, cs, cg, cc0, cc1, co,
                  s_sh, g_sh, gsem, ssem, zsem):
    cidx = lax.axis_index("c")
    sid = lax.axis_index("s")
    r0 = sid * RPT

    pltpu.sync_copy(row_hbm.at[sid], row_all)
    pltpu.sync_copy(col_hbm.at[sid], col_all)
    pltpu.sync_copy(g0_hbm.at[cidx, pl.ds(r0, RPT)], g_sh.at[pl.ds(r0, RPT)])
    pltpu.sync_copy(zero_hbm.at[pl.ds(r0, RPT)], s_sh.at[pl.ds(r0, RPT)])
    plsc.subcore_barrier()

    def gather_start(j, b):
        pltpu.async_copy(g_sh.at[row_all.at[j]], rows.at[b], gsem.at[b])

    def gather_wait(b):
        pltpu.make_async_copy(g_sh.at[row_all.at[b]], rows.at[b],
                              gsem.at[b]).wait()

    def scatter_start(j, b):
        pltpu.async_copy(rows.at[b], s_sh.at[col_all.at[j]], ssem.at[b],
                         add=True)

    def scatter_wait(b):
        pltpu.make_async_copy(rows.at[b], s_sh.at[col_all.at[b]],
                              ssem.at[b]).wait()

    def one_iter(it, carry):
        for b in range(LOOKAHEAD):
            gather_start(b, b)

        def ring(i, carry2):
            jo = i * NBUF
            for b in range(NBUF):
                j = jo + b
                bg = (b + LOOKAHEAD) % NBUF
                gather_wait(b)
                scatter_start(j, b)

                @pl.when(j >= LOOKAHEAD)
                def _():
                    scatter_wait(bg)

                @pl.when(j + LOOKAHEAD < LOOP_CPT)
                def _():
                    gather_start(j + LOOKAHEAD, bg)
            return carry2

        lax.fori_loop(0, LOOP_CPT // NBUF, ring, 0)
        for b in range(NBUF - LOOKAHEAD, NBUF):
            scatter_wait(b)
        plsc.subcore_barrier()

        for q in range(NCMB):
            rq = r0 + q * CMB
            pltpu.sync_copy(s_sh.at[pl.ds(rq, CMB)], cs)
            pltpu.sync_copy(g_sh.at[pl.ds(rq, CMB)], cg)
            pltpu.sync_copy(c0_hbm.at[cidx, pl.ds(rq, CMB)], cc0)
            pltpu.sync_copy(c1_hbm.at[pl.ds(rq, CMB)], cc1)
            pltpu.async_copy(zero_hbm.at[pl.ds(rq, CMB)],
                             s_sh.at[pl.ds(rq, CMB)], zsem)

            def vrow(i, carry3):
                for off in (0, 8):
                    sl = pl.ds(off, 16)
                    co[i, sl] = (cc1[i, sl] * (cs[i, sl] + cg[i, sl])
                                 + cc0[i, sl])
                return carry3

            lax.fori_loop(0, CMB, vrow, 0)
            pltpu.sync_copy(co, g_sh.at[pl.ds(rq, CMB)])

        for q in range(NCMB):
            pltpu.make_async_copy(zero_hbm.at[pl.ds(r0, CMB)],
                                  s_sh.at[pl.ds(r0, CMB)], zsem).wait()
        plsc.subcore_barrier()
        return carry

    lax.fori_loop(0, K, one_iter, 0)
    pltpu.sync_copy(g_sh.at[pl.ds(r0, RPT)], g_out.at[cidx, pl.ds(r0, RPT)])


_sc_loop = functools.partial(
    pl.kernel,
    out_type=jax.ShapeDtypeStruct((2, NP, CH), jnp.float32),
    mesh=plsc.VectorSubcoreMesh(core_axis_name="c", subcore_axis_name="s"),
    scratch_types=[
        pltpu.VMEM((LOOP_CPT, CHUNK), jnp.int32),
        pltpu.VMEM((LOOP_CPT, CHUNK), jnp.int32),
        pltpu.VMEM((NBUF, CHUNK, CH), jnp.float32),
        pltpu.VMEM((CMB, CH), jnp.float32),
        pltpu.VMEM((CMB, CH), jnp.float32),
        pltpu.VMEM((CMB, CH), jnp.float32),
        pltpu.VMEM((CMB, CH), jnp.float32),
        pltpu.VMEM((CMB, CH), jnp.float32),
        pltpu.VMEM_SHARED((NP, CH), jnp.float32),
        pltpu.VMEM_SHARED((NP, CH), jnp.float32),
        pltpu.SemaphoreType.DMA((NBUF,)),
        pltpu.SemaphoreType.DMA((NBUF,)),
        pltpu.SemaphoreType.DMA,
    ],
    compiler_params=pltpu.CompilerParams(use_tc_tiling_on_sc=False),
)(_sc_loop_body)


_BLK = 128
_GRID = NP // _BLK

def _row_spec():
    return pl.BlockSpec((_BLK, CP), lambda i: (i, 0))


def _lin_body(x_ref, w_ref, b_ref, o_ref):
    acc = jnp.dot(x_ref[...], w_ref[...], preferred_element_type=jnp.float32)
    o_ref[...] = jnp.maximum(acc + b_ref[...], 0.0)


def _tc_linear(x_pad, w_pad, b_pad):
    return pl.pallas_call(
        _lin_body,
        grid=(_GRID,),
        in_specs=[
            pl.BlockSpec((_BLK, D), lambda i: (i, 0)),
            pl.BlockSpec((D, CP), lambda i: (0, 0)),
            pl.BlockSpec((1, CP), lambda i: (0, 0)),
        ],
        out_specs=_row_spec(),
        out_shape=jax.ShapeDtypeStruct((NP, CP), jnp.float32),
    )(x_pad, w_pad, b_pad)


def _init_body(d0_ref, d1_ref, h0_ref, g_ref, c0_ref, c1_ref, dinv_ref):
    deg = d0_ref[...][:, 0:1] + d1_ref[...][:, 0:1] + 1.0
    dinv = lax.rsqrt(deg)
    g = dinv * h0_ref[...]
    g_ref[...] = g
    c0_ref[...] = ALPHA * g
    c1_ref[...] = jnp.broadcast_to((1.0 - ALPHA) * dinv * dinv, (_BLK, CP))
    dinv_ref[...] = jnp.broadcast_to(dinv, (_BLK, CP))


def _tc_init(s_deg, h0):
    dspec = pl.BlockSpec((_BLK, 16), lambda i: (i, 0))
    return pl.pallas_call(
        _init_body,
        grid=(_GRID,),
        in_specs=[dspec, dspec, _row_spec()],
        out_specs=[_row_spec(), _row_spec(), _row_spec(), _row_spec()],
        out_shape=[jax.ShapeDtypeStruct((NP, CP), jnp.float32)] * 4,
    )(s_deg[0], s_deg[1], h0)


def _final_body(g_ref, dinv_ref, o_ref):
    h = g_ref[...] / dinv_ref[...]
    col = lax.broadcasted_iota(jnp.int32, (_BLK, CP), 1)
    hm = jnp.where(col < C, h, -1e30)
    m = jnp.max(hm, axis=1, keepdims=True)
    e = jnp.exp(hm - m)
    lse = jnp.log(jnp.sum(e, axis=1, keepdims=True)) + m
    o_ref[...] = hm - lse


def _tc_final(g48, dinv):
    return pl.pallas_call(
        _final_body,
        grid=(_GRID,),
        in_specs=[_row_spec(), _row_spec()],
        out_specs=_row_spec(),
        out_shape=jax.ShapeDtypeStruct((NP, CP), jnp.float32),
    )(g48, dinv)


def kernel(x, edge_index, W, b):
    x_pad = jnp.pad(x, ((0, NP - N), (0, 0)))
    w_pad = jnp.pad(W, ((0, 0), (0, CP - C)))
    b_pad = jnp.pad(b, (0, CP - C)).reshape(1, CP)

    e = edge_index.shape[1]
    row_flat = jnp.pad(edge_index[0], (0, EP - e), constant_values=DUMMY)
    col_flat = jnp.pad(edge_index[1], (0, EP - e), constant_values=DUMMY)
    col_deg = col_flat.reshape(DEG_TILES, DEG_CPT, CHUNK)
    row2 = row_flat.reshape(16, LOOP_CPT, CHUNK)
    col2 = col_flat.reshape(16, LOOP_CPT, CHUNK)

    ones16 = jnp.ones((CHUNK, 16), jnp.float32)
    zero16 = jnp.zeros((NP, 16), jnp.float32)
    zero24 = jnp.zeros((NP, CH), jnp.float32)

    h0 = _tc_linear(x_pad, w_pad, b_pad)
    s_deg = _sc_degrees(col_deg, ones16, zero16)
    g0, c0, c1, dinv = _tc_init(s_deg, h0)

    g0h = jnp.stack([g0[:, :CH], g0[:, CH:]])
    c0h = jnp.stack([c0[:, :CH], c0[:, CH:]])
    c1h = c1[:, :CH]

    g_fin = _sc_loop(g0h, c0h, c1h, row2, col2, zero24)
    g48 = jnp.concatenate([g_fin[0], g_fin[1]], axis=1)

    z = _tc_final(g48, dinv)
    return z[:N, :C]


# speedup vs baseline: 33.1607x; 1.0395x over previous
import functools

import jax
import jax.numpy as jnp
from jax import lax
from jax.experimental import pallas as pl
from jax.experimental.pallas import tpu as pltpu
from jax.experimental.pallas import tpu_sc as plsc

N = 10000
D = 128
C = 40
K = 10
ALPHA = 0.1

NP = 10112
CP = 48
CH = CP // 2
DUMMY = 10000

CHUNK = 128
EP = 327680

DEG_TILES = 32
DEG_CPT = 80
LOOP_CPT = 160

NBUF = 8
LOOKAHEAD = 4
DEG_NBUF = 8

RPT = NP // 16
CMB = 79
NCMB = RPT // CMB


def _sc_deg_body(col_hbm, ones_hbm, zero_hbm, out_hbm,
                 col_all, ones_v, s_sh, ssem):
    c = lax.axis_index("c")
    sid = lax.axis_index("s")
    t = c * 16 + sid
    r0 = sid * RPT

    pltpu.sync_copy(col_hbm.at[t], col_all)
    pltpu.sync_copy(ones_hbm, ones_v)
    pltpu.sync_copy(zero_hbm.at[pl.ds(r0, RPT)], s_sh.at[pl.ds(r0, RPT)])
    plsc.subcore_barrier()

    def fire(j):
        pltpu.async_copy(ones_v, s_sh.at[col_all.at[j]], ssem, add=True)

    def drain_one():
        pltpu.make_async_copy(ones_v, s_sh.at[col_all.at[0]], ssem).wait()

    for b in range(DEG_NBUF):
        fire(b)

    def outer(i, carry):
        jo = i * DEG_NBUF
        for b in range(DEG_NBUF):
            drain_one()

            @pl.when(jo + DEG_NBUF + b < DEG_CPT)
            def _():
                fire(jo + DEG_NBUF + b)
        return carry

    lax.fori_loop(0, DEG_CPT // DEG_NBUF, outer, 0)

    plsc.subcore_barrier()
    pltpu.sync_copy(s_sh.at[pl.ds(r0, RPT)], out_hbm.at[c, pl.ds(r0, RPT)])


_sc_degrees = functools.partial(
    pl.kernel,
    out_type=jax.ShapeDtypeStruct((2, NP, 16), jnp.float32),
    mesh=plsc.VectorSubcoreMesh(core_axis_name="c", subcore_axis_name="s"),
    scratch_types=[
        pltpu.VMEM((DEG_CPT, CHUNK), jnp.int32),
        pltpu.VMEM((CHUNK, 16), jnp.float32),
        pltpu.VMEM_SHARED((NP, 16), jnp.float32),
        pltpu.SemaphoreType.DMA,
    ],
    compiler_params=pltpu.CompilerParams(use_tc_tiling_on_sc=False),
)(_sc_deg_body)


def _sc_loop_body(g0_hbm, c0_hbm, c1_hbm, row_hbm, col_hbm, zero_hbm,
                  g_out,
                  row_all, col_all, rows, cs, cg, cc0a, cc1a,
                  s_sh, g_sh, gsem, ssem, zsem):
    cidx = lax.axis_index("c")
    sid = lax.axis_index("s")
    r0 = sid * RPT

    pltpu.sync_copy(row_hbm.at[sid], row_all)
    pltpu.sync_copy(col_hbm.at[sid], col_all)
    pltpu.sync_copy(g0_hbm.at[cidx, pl.ds(r0, RPT)], g_sh.at[pl.ds(r0, RPT)])
    pltpu.sync_copy(zero_hbm.at[pl.ds(r0, RPT)], s_sh.at[pl.ds(r0, RPT)])
    pltpu.sync_copy(c0_hbm.at[cidx, pl.ds(r0, RPT)], cc0a)
    pltpu.sync_copy(c1_hbm.at[pl.ds(r0, RPT)], cc1a)
    plsc.subcore_barrier()

    def gather_start(j, b):
        pltpu.async_copy(g_sh.at[row_all.at[j]], rows.at[b], gsem.at[b])

    def gather_wait(b):
        pltpu.make_async_copy(g_sh.at[row_all.at[b]], rows.at[b],
                              gsem.at[b]).wait()

    def scatter_start(j, b):
        pltpu.async_copy(rows.at[b], s_sh.at[col_all.at[j]], ssem.at[b],
                         add=True)

    def scatter_wait(b):
        pltpu.make_async_copy(rows.at[b], s_sh.at[col_all.at[b]],
                              ssem.at[b]).wait()

    def one_iter(it, carry):
        for b in range(LOOKAHEAD):
            gather_start(b, b)

        def ring(i, carry2):
            jo = i * NBUF
            for b in range(NBUF):
                j = jo + b
                bg = (b + LOOKAHEAD) % NBUF
                gather_wait(b)
                scatter_start(j, b)

                @pl.when(j >= LOOKAHEAD)
                def _():
                    scatter_wait(bg)

                @pl.when(j + LOOKAHEAD < LOOP_CPT)
                def _():
                    gather_start(j + LOOKAHEAD, bg)
            return carry2

        lax.fori_loop(0, LOOP_CPT // NBUF, ring, 0)
        for b in range(NBUF - LOOKAHEAD, NBUF):
            scatter_wait(b)
        plsc.subcore_barrier()

        for q in range(NCMB):
            rq = r0 + q * CMB
            pltpu.sync_copy(s_sh.at[pl.ds(rq, CMB)], cs)
            pltpu.sync_copy(g_sh.at[pl.ds(rq, CMB)], cg)
            pltpu.async_copy(zero_hbm.at[pl.ds(rq, CMB)],
                             s_sh.at[pl.ds(rq, CMB)], zsem)

            def vrow(i, carry3):
                i0 = q * CMB + i
                sl0 = pl.ds(0, 16)
                sl8 = pl.ds(8, 16)
                a0 = (cc1a[i0, sl0] * (cs[i, sl0] + cg[i, sl0])
                      + cc0a[i0, sl0])
                a8 = (cc1a[i0, sl8] * (cs[i, sl8] + cg[i, sl8])
                      + cc0a[i0, sl8])
                cg[i, sl0] = a0
                cg[i, sl8] = a8
                return carry3

            lax.fori_loop(0, CMB, vrow, 0)
            pltpu.sync_copy(cg, g_sh.at[pl.ds(rq, CMB)])

        for q in range(NCMB):
            pltpu.make_async_copy(zero_hbm.at[pl.ds(r0, CMB)],
                                  s_sh.at[pl.ds(r0, CMB)], zsem).wait()
        plsc.subcore_barrier()
        return carry

    lax.fori_loop(0, K, one_iter, 0)
    pltpu.sync_copy(g_sh.at[pl.ds(r0, RPT)], g_out.at[cidx, pl.ds(r0, RPT)])


_sc_loop = functools.partial(
    pl.kernel,
    out_type=jax.ShapeDtypeStruct((2, NP, CH), jnp.float32),
    mesh=plsc.VectorSubcoreMesh(core_axis_name="c", subcore_axis_name="s"),
    scratch_types=[
        pltpu.VMEM((LOOP_CPT, CHUNK), jnp.int32),
        pltpu.VMEM((LOOP_CPT, CHUNK), jnp.int32),
        pltpu.VMEM((NBUF, CHUNK, CH), jnp.float32),
        pltpu.VMEM((CMB, CH), jnp.float32),
        pltpu.VMEM((CMB, CH), jnp.float32),
        pltpu.VMEM((RPT, CH), jnp.float32),
        pltpu.VMEM((RPT, CH), jnp.float32),
        pltpu.VMEM_SHARED((NP, CH), jnp.float32),
        pltpu.VMEM_SHARED((NP, CH), jnp.float32),
        pltpu.SemaphoreType.DMA((NBUF,)),
        pltpu.SemaphoreType.DMA((NBUF,)),
        pltpu.SemaphoreType.DMA,
    ],
    compiler_params=pltpu.CompilerParams(use_tc_tiling_on_sc=False),
)(_sc_loop_body)


_BLK = 128
_GRID = NP // _BLK

def _row_spec():
    return pl.BlockSpec((_BLK, CP), lambda i: (i, 0))


def _lin_body(x_ref, w_ref, b_ref, o_ref):
    acc = jnp.dot(x_ref[...], w_ref[...], preferred_element_type=jnp.float32)
    o_ref[...] = jnp.maximum(acc + b_ref[...], 0.0)


def _tc_linear(x_pad, w_pad, b_pad):
    return pl.pallas_call(
        _lin_body,
        grid=(_GRID,),
        in_specs=[
            pl.BlockSpec((_BLK, D), lambda i: (i, 0)),
            pl.BlockSpec((D, CP), lambda i: (0, 0)),
            pl.BlockSpec((1, CP), lambda i: (0, 0)),
        ],
        out_specs=_row_spec(),
        out_shape=jax.ShapeDtypeStruct((NP, CP), jnp.float32),
    )(x_pad, w_pad, b_pad)


def _init_body(d0_ref, d1_ref, h0_ref, g_ref, c0_ref, c1_ref, dinv_ref):
    deg = d0_ref[...][:, 0:1] + d1_ref[...][:, 0:1] + 1.0
    dinv = lax.rsqrt(deg)
    g = dinv * h0_ref[...]
    g_ref[...] = g
    c0_ref[...] = ALPHA * g
    c1_ref[...] = jnp.broadcast_to((1.0 - ALPHA) * dinv * dinv, (_BLK, CP))
    dinv_ref[...] = jnp.broadcast_to(dinv, (_BLK, CP))


def _tc_init(s_deg, h0):
    dspec = pl.BlockSpec((_BLK, 16), lambda i: (i, 0))
    return pl.pallas_call(
        _init_body,
        grid=(_GRID,),
        in_specs=[dspec, dspec, _row_spec()],
        out_specs=[_row_spec(), _row_spec(), _row_spec(), _row_spec()],
        out_shape=[jax.ShapeDtypeStruct((NP, CP), jnp.float32)] * 4,
    )(s_deg[0], s_deg[1], h0)


def _final_body(g_ref, dinv_ref, o_ref):
    h = g_ref[...] / dinv_ref[...]
    col = lax.broadcasted_iota(jnp.int32, (_BLK, CP), 1)
    hm = jnp.where(col < C, h, -1e30)
    m = jnp.max(hm, axis=1, keepdims=True)
    e = jnp.exp(hm - m)
    lse = jnp.log(jnp.sum(e, axis=1, keepdims=True)) + m
    o_ref[...] = hm - lse


def _tc_final(g48, dinv):
    return pl.pallas_call(
        _final_body,
        grid=(_GRID,),
        in_specs=[_row_spec(), _row_spec()],
        out_specs=_row_spec(),
        out_shape=jax.ShapeDtypeStruct((NP, CP), jnp.float32),
    )(g48, dinv)


def kernel(x, edge_index, W, b):
    x_pad = jnp.pad(x, ((0, NP - N), (0, 0)))
    w_pad = jnp.pad(W, ((0, 0), (0, CP - C)))
    b_pad = jnp.pad(b, (0, CP - C)).reshape(1, CP)

    e = edge_index.shape[1]
    row_flat = jnp.pad(edge_index[0], (0, EP - e), constant_values=DUMMY)
    col_flat = jnp.pad(edge_index[1], (0, EP - e), constant_values=DUMMY)
    col_deg = col_flat.reshape(DEG_TILES, DEG_CPT, CHUNK)
    row2 = row_flat.reshape(16, LOOP_CPT, CHUNK)
    col2 = col_flat.reshape(16, LOOP_CPT, CHUNK)

    ones16 = jnp.ones((CHUNK, 16), jnp.float32)
    zero16 = jnp.zeros((NP, 16), jnp.float32)
    zero24 = jnp.zeros((NP, CH), jnp.float32)

    h0 = _tc_linear(x_pad, w_pad, b_pad)
    s_deg = _sc_degrees(col_deg, ones16, zero16)
    g0, c0, c1, dinv = _tc_init(s_deg, h0)

    g0h = jnp.stack([g0[:, :CH], g0[:, CH:]])
    c0h = jnp.stack([c0[:, :CH], c0[:, CH:]])
    c1h = c1[:, :CH]

    g_fin = _sc_loop(g0h, c0h, c1h, row2, col2, zero24)
    g48 = jnp.concatenate([g_fin[0], g_fin[1]], axis=1)

    z = _tc_final(g48, dinv)
    return z[:N, :C]


# speedup vs baseline: 34.0586x; 1.0271x over previous
import functools

import jax
import jax.numpy as jnp
from jax import lax
from jax.experimental import pallas as pl
from jax.experimental.pallas import tpu as pltpu
from jax.experimental.pallas import tpu_sc as plsc

N = 10000
D = 128
C = 40
K = 10
ALPHA = 0.1

NP = 10112
CP = 48
CH = CP // 2
DUMMY = 10000

CHUNK = 128
EP = 327680

DEG_TILES = 32
DEG_CPT = 80
LOOP_CPT = 160

NBUF = 8
LOOKAHEAD = 4
DEG_NBUF = 8

RPT = NP // 16
CMB = 79
NCMB = RPT // CMB


def _sc_deg_body(col_hbm, ones_hbm, zero_hbm, out_hbm,
                 col_all, ones_v, s_sh, ssem):
    c = lax.axis_index("c")
    sid = lax.axis_index("s")
    t = c * 16 + sid
    r0 = sid * RPT

    pltpu.sync_copy(col_hbm.at[t], col_all)
    pltpu.sync_copy(ones_hbm, ones_v)
    pltpu.sync_copy(zero_hbm.at[pl.ds(r0, RPT)], s_sh.at[pl.ds(r0, RPT)])
    plsc.subcore_barrier()

    def fire(j):
        pltpu.async_copy(ones_v, s_sh.at[col_all.at[j]], ssem, add=True)

    def drain_one():
        pltpu.make_async_copy(ones_v, s_sh.at[col_all.at[0]], ssem).wait()

    for b in range(DEG_NBUF):
        fire(b)

    def outer(i, carry):
        jo = i * DEG_NBUF
        for b in range(DEG_NBUF):
            drain_one()

            @pl.when(jo + DEG_NBUF + b < DEG_CPT)
            def _():
                fire(jo + DEG_NBUF + b)
        return carry

    lax.fori_loop(0, DEG_CPT // DEG_NBUF, outer, 0)

    plsc.subcore_barrier()
    pltpu.sync_copy(s_sh.at[pl.ds(r0, RPT)], out_hbm.at[c, pl.ds(r0, RPT)])


_sc_degrees = functools.partial(
    pl.kernel,
    out_type=jax.ShapeDtypeStruct((2, NP, 16), jnp.float32),
    mesh=plsc.VectorSubcoreMesh(core_axis_name="c", subcore_axis_name="s"),
    scratch_types=[
        pltpu.VMEM((DEG_CPT, CHUNK), jnp.int32),
        pltpu.VMEM((CHUNK, 16), jnp.float32),
        pltpu.VMEM_SHARED((NP, 16), jnp.float32),
        pltpu.SemaphoreType.DMA,
    ],
    compiler_params=pltpu.CompilerParams(use_tc_tiling_on_sc=False),
)(_sc_deg_body)


def _sc_loop_body(g0_hbm, c0_hbm, c1_hbm, row_hbm, col_hbm, zero_hbm,
                  g_out,
                  row_all, col_all, rows, cs, cg, cc0a, cc1a,
                  s_sh, g_sh, gsem, ssem, zsem):
    cidx = lax.axis_index("c")
    sid = lax.axis_index("s")
    r0 = sid * RPT

    pltpu.sync_copy(row_hbm.at[sid], row_all)
    pltpu.sync_copy(col_hbm.at[sid], col_all)
    pltpu.sync_copy(g0_hbm.at[cidx, pl.ds(r0, RPT)], g_sh.at[pl.ds(r0, RPT)])
    pltpu.sync_copy(zero_hbm.at[pl.ds(r0, RPT)], s_sh.at[pl.ds(r0, RPT)])
    pltpu.sync_copy(c0_hbm.at[cidx, pl.ds(r0, RPT)], cc0a)
    pltpu.sync_copy(c1_hbm.at[pl.ds(r0, RPT)], cc1a)
    plsc.subcore_barrier()

    def gather_start(j, b):
        pltpu.async_copy(g_sh.at[row_all.at[j]], rows.at[b], gsem.at[b])

    def gather_wait(b):
        pltpu.make_async_copy(g_sh.at[row_all.at[b]], rows.at[b],
                              gsem.at[b]).wait()

    def scatter_start(j, b):
        pltpu.async_copy(rows.at[b], s_sh.at[col_all.at[j]], ssem.at[b],
                         add=True)

    def scatter_wait(b):
        pltpu.make_async_copy(rows.at[b], s_sh.at[col_all.at[b]],
                              ssem.at[b]).wait()

    def one_iter(it, carry):
        for b in range(LOOKAHEAD):
            gather_start(b, b)

        def ring(i, carry2):
            jo = i * NBUF
            for b in range(NBUF):
                j = jo + b
                bg = (b + LOOKAHEAD) % NBUF
                gather_wait(b)
                scatter_start(j, b)

                @pl.when(j >= LOOKAHEAD)
                def _():
                    scatter_wait(bg)

                @pl.when(j + LOOKAHEAD < LOOP_CPT)
                def _():
                    gather_start(j + LOOKAHEAD, bg)
            return carry2

        lax.fori_loop(0, LOOP_CPT // NBUF, ring, 0)
        for b in range(NBUF - LOOKAHEAD, NBUF):
            scatter_wait(b)
        plsc.subcore_barrier()

        for q in range(NCMB):
            rq = r0 + q * CMB
            pltpu.sync_copy(s_sh.at[pl.ds(rq, CMB)], cs)
            pltpu.sync_copy(g_sh.at[pl.ds(rq, CMB)], cg)
            pltpu.async_copy(zero_hbm.at[pl.ds(rq, CMB)],
                             s_sh.at[pl.ds(rq, CMB)], zsem)

            def vrow(i, carry3):
                i0 = q * CMB + i
                sl0 = pl.ds(0, 16)
                sl8 = pl.ds(8, 16)
                a0 = (cc1a[i0, sl0] * (cs[i, sl0] + cg[i, sl0])
                      + cc0a[i0, sl0])
                a8 = (cc1a[i0, sl8] * (cs[i, sl8] + cg[i, sl8])
                      + cc0a[i0, sl8])
                cg[i, sl0] = a0
                cg[i, sl8] = a8
                return carry3

            lax.fori_loop(0, CMB, vrow, 0)
            pltpu.sync_copy(cg, g_sh.at[pl.ds(rq, CMB)])

        for q in range(NCMB):
            pltpu.make_async_copy(zero_hbm.at[pl.ds(r0, CMB)],
                                  s_sh.at[pl.ds(r0, CMB)], zsem).wait()
        plsc.subcore_barrier()
        return carry

    lax.fori_loop(0, K, one_iter, 0)
    pltpu.sync_copy(g_sh.at[pl.ds(r0, RPT)], g_out.at[cidx, pl.ds(r0, RPT)])


_sc_loop = functools.partial(
    pl.kernel,
    out_type=jax.ShapeDtypeStruct((2, NP, CH), jnp.float32),
    mesh=plsc.VectorSubcoreMesh(core_axis_name="c", subcore_axis_name="s"),
    scratch_types=[
        pltpu.VMEM((LOOP_CPT, CHUNK), jnp.int32),
        pltpu.VMEM((LOOP_CPT, CHUNK), jnp.int32),
        pltpu.VMEM((NBUF, CHUNK, CH), jnp.float32),
        pltpu.VMEM((CMB, CH), jnp.float32),
        pltpu.VMEM((CMB, CH), jnp.float32),
        pltpu.VMEM((RPT, CH), jnp.float32),
        pltpu.VMEM((RPT, CH), jnp.float32),
        pltpu.VMEM_SHARED((NP, CH), jnp.float32),
        pltpu.VMEM_SHARED((NP, CH), jnp.float32),
        pltpu.SemaphoreType.DMA((NBUF,)),
        pltpu.SemaphoreType.DMA((NBUF,)),
        pltpu.SemaphoreType.DMA,
    ],
    compiler_params=pltpu.CompilerParams(use_tc_tiling_on_sc=False),
)(_sc_loop_body)


_BLK = 128
_GRID = NP // _BLK

def _row_spec():
    return pl.BlockSpec((_BLK, CP), lambda i: (i, 0))


def _lin_body(x_ref, w_ref, b_ref, o_ref):
    acc = jnp.dot(x_ref[...], w_ref[...], preferred_element_type=jnp.float32)
    o_ref[...] = jnp.maximum(acc + b_ref[...], 0.0)


def _tc_linear(x_pad, w_pad, b_pad):
    return pl.pallas_call(
        _lin_body,
        grid=(_GRID,),
        in_specs=[
            pl.BlockSpec((_BLK, D), lambda i: (i, 0)),
            pl.BlockSpec((D, CP), lambda i: (0, 0)),
            pl.BlockSpec((1, CP), lambda i: (0, 0)),
        ],
        out_specs=_row_spec(),
        out_shape=jax.ShapeDtypeStruct((NP, CP), jnp.float32),
    )(x_pad, w_pad, b_pad)


def _init_body(d0_ref, d1_ref, h0_ref, g_ref, c0_ref, c1_ref, dinv_ref):
    deg = d0_ref[...][:, 0:1] + d1_ref[...][:, 0:1] + 1.0
    dinv = lax.rsqrt(deg)
    g = dinv * h0_ref[...]
    g_ref[0] = g[:, :CH]
    g_ref[1] = g[:, CH:]
    c0_ref[0] = ALPHA * g[:, :CH]
    c0_ref[1] = ALPHA * g[:, CH:]
    c1_ref[...] = jnp.broadcast_to((1.0 - ALPHA) * dinv * dinv, (_BLK, CH))
    dinv_ref[...] = jnp.broadcast_to(dinv, (_BLK, CP))


def _tc_init(s_deg, h0):
    dspec = pl.BlockSpec((_BLK, 16), lambda i: (i, 0))
    hspec = pl.BlockSpec((2, _BLK, CH), lambda i: (0, i, 0))
    return pl.pallas_call(
        _init_body,
        grid=(_GRID,),
        in_specs=[dspec, dspec, _row_spec()],
        out_specs=[hspec, hspec,
                   pl.BlockSpec((_BLK, CH), lambda i: (i, 0)), _row_spec()],
        out_shape=[jax.ShapeDtypeStruct((2, NP, CH), jnp.float32),
                   jax.ShapeDtypeStruct((2, NP, CH), jnp.float32),
                   jax.ShapeDtypeStruct((NP, CH), jnp.float32),
                   jax.ShapeDtypeStruct((NP, CP), jnp.float32)],
    )(s_deg[0], s_deg[1], h0)


def _final_body(g_ref, dinv_ref, o_ref):
    g = jnp.concatenate([g_ref[0], g_ref[1]], axis=1)
    h = g / dinv_ref[...]
    col = lax.broadcasted_iota(jnp.int32, (_BLK, CP), 1)
    hm = jnp.where(col < C, h, -1e30)
    m = jnp.max(hm, axis=1, keepdims=True)
    e = jnp.exp(hm - m)
    lse = jnp.log(jnp.sum(e, axis=1, keepdims=True)) + m
    o_ref[...] = hm - lse


def _tc_final(g_fin, dinv):
    return pl.pallas_call(
        _final_body,
        grid=(_GRID,),
        in_specs=[pl.BlockSpec((2, _BLK, CH), lambda i: (0, i, 0)),
                  _row_spec()],
        out_specs=_row_spec(),
        out_shape=jax.ShapeDtypeStruct((NP, CP), jnp.float32),
    )(g_fin, dinv)


def kernel(x, edge_index, W, b):
    x_pad = jnp.pad(x, ((0, NP - N), (0, 0)))
    w_pad = jnp.pad(W, ((0, 0), (0, CP - C)))
    b_pad = jnp.pad(b, (0, CP - C)).reshape(1, CP)

    e = edge_index.shape[1]
    row_flat = jnp.pad(edge_index[0], (0, EP - e), constant_values=DUMMY)
    col_flat = jnp.pad(edge_index[1], (0, EP - e), constant_values=DUMMY)
    col_deg = col_flat.reshape(DEG_TILES, DEG_CPT, CHUNK)
    row2 = row_flat.reshape(16, LOOP_CPT, CHUNK)
    col2 = col_flat.reshape(16, LOOP_CPT, CHUNK)

    ones16 = jnp.ones((CHUNK, 16), jnp.float32)
    zero16 = jnp.zeros((NP, 16), jnp.float32)
    zero24 = jnp.zeros((NP, CH), jnp.float32)

    h0 = _tc_linear(x_pad, w_pad, b_pad)
    s_deg = _sc_degrees(col_deg, ones16, zero16)
    g0h, c0h, c1h, dinv = _tc_init(s_deg, h0)

    g_fin = _sc_loop(g0h, c0h, c1h, row2, col2, zero24)

    z = _tc_final(g_fin, dinv)
    return z[:N, :C]
